# baseline (speedup 1.0000x reference)
import functools

import jax
import jax.numpy as jnp
from jax import lax
from jax.experimental import pallas as pl
from jax.experimental.pallas import tpu as pltpu
from jax.experimental.pallas import tpu_sc as plsc

_NUM_EMBEDDINGS = 1000000
_DIM = 64
_BATCH = 16384
_HIST = 200
_B = _BATCH * _HIST

_NC = 2
_NS = 16
_NW = _NC * _NS

_B_PER_W = _B // _NW
_CHUNK = 400
_ROWS_PER_CHUNK = _CHUNK // _HIST
_BROWS_PER_W = _BATCH // _NW
_STEPS = _B_PER_W // _CHUNK
_NBUF = 4
_G = _STEPS // _NBUF


def _body(table_hbm, idx_hbm, out_hbm, idx_v, rows_v,
          isem0, isem1, isem2, isem3,
          gsem0, gsem1, gsem2, gsem3,
          osem0, osem1, osem2, osem3):
    wid = lax.axis_index("s") * _NC + lax.axis_index("c")
    base = wid * _B_PER_W
    isems = (isem0, isem1, isem2, isem3)
    gsems = (gsem0, gsem1, gsem2, gsem3)
    osems = (osem0, osem1, osem2, osem3)

    def start_idx(i, b):
        off = base + i * _CHUNK
        pltpu.async_copy(idx_hbm.at[pl.ds(off, _CHUNK)], idx_v.at[b], isems[b])

    def wait_idx(b):
        pltpu.make_async_copy(idx_hbm.at[pl.ds(0, _CHUNK)], idx_v.at[b],
                              isems[b]).wait()

    def start_gather(b):
        for j in range(_ROWS_PER_CHUNK):
            pltpu.async_copy(
                table_hbm.at[idx_v.at[b, pl.ds(j * _HIST, _HIST)]],
                rows_v.at[b, j], gsems[b])

    def wait_gather(b):
        for j in range(_ROWS_PER_CHUNK):
            pltpu.make_async_copy(
                table_hbm.at[idx_v.at[b, pl.ds(j * _HIST, _HIST)]],
                rows_v.at[b, j], gsems[b]).wait()

    def start_write(i, b):
        boff = wid * _BROWS_PER_W + i * _ROWS_PER_CHUNK
        pltpu.async_copy(
            rows_v.at[b],
            out_hbm.at[pl.ds(boff, _ROWS_PER_CHUNK), :, pl.ds(0, _DIM)],
            osems[b])

    def wait_write(b):
        pltpu.make_async_copy(
            rows_v.at[b],
            out_hbm.at[pl.ds(0, _ROWS_PER_CHUNK), :, pl.ds(0, _DIM)],
            osems[b]).wait()

    def chunk_step(i, b):
        @pl.when(i > 1)
        def _():
            wait_gather((b - 2) % _NBUF)
            start_write(i - 2, (b - 2) % _NBUF)

        @pl.when(i + 2 < _STEPS)
        def _():
            start_idx(i + 2, (b + 2) % _NBUF)

        wait_idx(b)

        @pl.when(i > 3)
        def _():
            wait_write(b)

        start_gather(b)

    start_idx(0, 0)
    start_idx(1, 1)

    def gstep(g, carry):
        i0 = _NBUF * g
        for b in range(_NBUF):
            chunk_step(i0 + b, b)
        return carry

    lax.fori_loop(0, _G, gstep, 0)

    wait_gather((_STEPS - 2) % _NBUF)
    start_write(_STEPS - 2, (_STEPS - 2) % _NBUF)
    wait_gather((_STEPS - 1) % _NBUF)
    start_write(_STEPS - 1, (_STEPS - 1) % _NBUF)
    for b in range(_NBUF):
        wait_write(b)


_gather = functools.partial(
    pl.kernel,
    out_type=jax.ShapeDtypeStruct((_BATCH, _HIST, 2 * _DIM), jnp.float32),
    mesh=plsc.VectorSubcoreMesh(core_axis_name="c", subcore_axis_name="s"),
    scratch_types=[
        pltpu.VMEM((_NBUF, _CHUNK), jnp.int32),
        pltpu.VMEM((_NBUF, _ROWS_PER_CHUNK, _HIST, _DIM), jnp.float32),
    ] + [pltpu.SemaphoreType.DMA] * 12,
    compiler_params=pltpu.CompilerParams(use_tc_tiling_on_sc=False),
)(_body)


@jax.jit
def kernel(token_ids, weights):
    flat = token_ids.reshape(_B)
    padded = _gather(weights, flat)
    return padded[:, :, :_DIM]
